# Initial kernel scaffold; baseline (speedup 1.0000x reference)
import jax
import jax.numpy as jnp
from jax.experimental import pallas as pl


def kernel(x, y, x_pos, y_pos):
    raise NotImplementedError("write your pallas kernel here")


# speedup vs baseline: 2447.5809x; 2447.5809x over previous
import functools

import jax
import jax.numpy as jnp
from jax import lax
from jax.experimental import pallas as pl
from jax.experimental.pallas import tpu as pltpu


def _wasserstein_body(n_u, n_v, xp_ref, yp_ref, xw_ref, yw_ref, o_ref):
    n = n_u + n_v
    xw = xw_ref[...]
    yw = yw_ref[...]
    usum = jnp.sum(xw, axis=0, keepdims=True)
    vsum = jnp.sum(yw, axis=0, keepdims=True)
    keys = jnp.concatenate([xp_ref[...], yp_ref[...]], axis=0)
    vals = jnp.concatenate([xw / usum, -(yw / vsum)], axis=0)
    r = keys.shape[1]

    m = 2
    while m <= n:
        d = m // 2
        while d >= 1:
            g = n // (2 * d)
            k3 = keys.reshape(g, 2 * d, r)
            v3 = vals.reshape(g, 2 * d, r)
            ka, kb = k3[:, :d, :], k3[:, d:, :]
            va, vb = v3[:, :d, :], v3[:, d:, :]
            swap = ka > kb
            if m < n:
                desc = (
                    lax.broadcasted_iota(jnp.int32, (g, 1, 1), 0)
                    // (m // (2 * d))
                ) % 2 == 1
                swap = swap != desc
            lo_k = jnp.where(swap, kb, ka)
            hi_k = jnp.where(swap, ka, kb)
            lo_v = jnp.where(swap, vb, va)
            hi_v = jnp.where(swap, va, vb)
            keys = jnp.concatenate([lo_k, hi_k], axis=1).reshape(n, r)
            vals = jnp.concatenate([lo_v, hi_v], axis=1).reshape(n, r)
            d //= 2
        m *= 2

    c = vals
    sh = 1
    while sh < n:
        c = c + jnp.concatenate(
            [jnp.zeros((sh, r), jnp.float32), c[: n - sh, :]], axis=0
        )
        sh *= 2

    gaps = keys[1:, :] - keys[:-1, :]
    o_ref[...] = jnp.sum(jnp.abs(c[:-1, :]) * gaps, axis=0, keepdims=True)


def kernel(x, y, x_pos, y_pos):
    b, n_u = x.shape
    n_v = y.shape[1]
    xp_t = x_pos.T
    yp_t = y_pos.T
    xw_t = x.T
    yw_t = y.T
    r = min(128, b)
    grid = (b // r,)
    out = pl.pallas_call(
        functools.partial(_wasserstein_body, n_u, n_v),
        grid=grid,
        in_specs=[
            pl.BlockSpec((n_u, r), lambda i: (0, i)),
            pl.BlockSpec((n_v, r), lambda i: (0, i)),
            pl.BlockSpec((n_u, r), lambda i: (0, i)),
            pl.BlockSpec((n_v, r), lambda i: (0, i)),
        ],
        out_specs=pl.BlockSpec((1, r), lambda i: (0, i)),
        out_shape=jax.ShapeDtypeStruct((1, b), jnp.float32),
        compiler_params=pltpu.CompilerParams(
            vmem_limit_bytes=100 * 1024 * 1024,
        ),
    )(xp_t, yp_t, xw_t, yw_t)
    return out.reshape(b)
